# Initial kernel scaffold; baseline (speedup 1.0000x reference)
import jax
import jax.numpy as jnp
from jax.experimental import pallas as pl


def kernel(seq_batch, node_idx_batch, edge_index, emb_table, W_ih_l0, W_hh_l0, b_ih_l0, b_hh_l0, W_ih_l1, W_hh_l1, b_ih_l1, b_hh_l1, W_rel, W_root, b_gnn, W_fc1, b_fc1, W_fc2, b_fc2):
    raise NotImplementedError("write your pallas kernel here")


# speedup vs baseline: 6.4510x; 6.4510x over previous
import functools

import jax
import jax.numpy as jnp
from jax import lax
from jax.experimental import pallas as pl
from jax.experimental.pallas import tpu as pltpu
from jax.experimental.pallas import tpu_sc as plsc


def _sc_edge_filter(node_idx, edge_src, edge_dst, emb_table):
    info = plsc.get_sparse_core_info()
    NC, NS = info.num_cores, info.num_subcores
    NW = NC * NS
    E = edge_src.shape[0]
    N, D = emb_table.shape
    B = node_idx.shape[0]
    assert E % NW == 0 and (E // NW) % 8 == 0
    EPW = E // NW
    G = (EPW + 15) // 16

    mesh = plsc.VectorSubcoreMesh(core_axis_name="c", subcore_axis_name="s")

    @functools.partial(
        pl.kernel,
        out_type=(
            jax.ShapeDtypeStruct((NW, D, B), jnp.float32),
            jax.ShapeDtypeStruct((B, D), jnp.float32),
        ),
        mesh=mesh,
        compiler_params=pltpu.CompilerParams(needs_layout_passes=False,
                                             use_tc_tiling_on_sc=False),
        scratch_types=[
            pltpu.VMEM((N,), jnp.int32),
            pltpu.VMEM((G * 16,), jnp.int32),
            pltpu.VMEM((G * 16,), jnp.int32),
            pltpu.VMEM((B * D,), jnp.float32),
            pltpu.VMEM((16, D), jnp.float32),
            pltpu.VMEM((B,), jnp.int32),
            pltpu.VMEM((D, B), jnp.float32),
            pltpu.VMEM((B,), jnp.int32),
            pltpu.VMEM((B, D), jnp.float32),
            pltpu.SemaphoreType.DMA,
        ],
    )
    def k(nidx_hbm, src_hbm, dst_hbm, emb_hbm, part_out, bemb_out,
          invmap, dstb, srcb, accf, rows, nidxv, outp, tmpb, bembv, sem):
        cid = lax.axis_index("c")
        sid = lax.axis_index("s")
        wid = sid * NC + cid
        lane = lax.iota(jnp.int32, 16)

        base_e = wid * EPW
        pltpu.sync_copy(src_hbm.at[pl.ds(base_e, EPW)], srcb.at[pl.ds(0, EPW)])
        pltpu.sync_copy(dst_hbm.at[pl.ds(base_e, EPW)], dstb.at[pl.ds(0, EPW)])
        pltpu.sync_copy(nidx_hbm, nidxv)

        @pl.when(wid == 0)
        def _():
            pltpu.async_copy(emb_hbm.at[nidxv], bembv, sem).wait()
            pltpu.sync_copy(bembv, bemb_out)

        def mset(i, _):
            invmap[pl.ds(i * 16, 16)] = jnp.full((16,), -1, jnp.int32)
            return 0
        lax.fori_loop(0, N // 16, mset, 0)

        def aset(i, _):
            accf[pl.ds(i * 16, 16)] = jnp.zeros((16,), jnp.float32)
            return 0
        lax.fori_loop(0, (B * D) // 16, aset, 0)

        for bg in range(B // 16):
            idxv = nidxv[pl.ds(bg * 16, 16)]
            plsc.store_scatter(invmap, [idxv], bg * 16 + lane)

        def group(g, _):
            base = g * 16
            valid = (base + lane) < EPW
            dstv = jnp.where(valid, dstb[pl.ds(base, 16)], 0)
            slots = plsc.load_gather(invmap, [dstv])
            m = (slots >= 0) & valid
            cnt = jnp.sum(m.astype(jnp.int32))

            @pl.when(cnt > 0)
            def _():
                srcv = jnp.where(m, srcb[pl.ds(base, 16)], 0)
                slots0 = jnp.where(m, slots, 0)
                pltpu.async_copy(emb_hbm.at[srcv], rows, sem).wait()
                sbase = slots0 * D

                def cond(rem):
                    return jnp.sum(rem) > 0

                def body(rem):
                    remb = rem > 0
                    plsc.store_scatter(tmpb, [slots0], lane, mask=remb)
                    rb = plsc.load_gather(tmpb, [slots0])
                    sel = (rb == lane) & remb
                    for c in range(D):
                        colv = plsc.load_gather(
                            rows, [lane, jnp.full((16,), c, jnp.int32)])
                        plsc.addupdate_scatter(accf, [sbase + c], colv, mask=sel)
                    return jnp.where(sel, 0, rem)

                lax.while_loop(cond, body, m.astype(jnp.int32))
            return 0
        lax.fori_loop(0, G, group, 0)

        for bg in range(B // 16):
            nv = nidxv[pl.ds(bg * 16, 16)]
            slotv = plsc.load_gather(invmap, [nv])
            sb = slotv * D

            def remap(c, _):
                vals = plsc.load_gather(accf, [sb + c])
                outp[c, pl.ds(bg * 16, 16)] = vals
                return 0
            lax.fori_loop(0, D, remap, 0)
        pltpu.sync_copy(outp, part_out.at[wid])

    return k(node_idx, edge_src, edge_dst, emb_table)


def _tc_fused(seq_t, partials, bemb,
              Wih0, Whh0, bi0, bh0, Wih1, Whh1, bi1, bh1,
              Wrel, Wroot, bgn, Wf1, bf1, Wf2, bf2):
    T, B, IN = seq_t.shape
    H = Whh0.shape[1]

    def body(seq_ref, part_ref, bemb_ref,
             wih0, whh0, rbi0, rbh0, wih1, whh1, rbi1, rbh1,
             wrel, wroot, rbg, wf1, rbf1, wf2, rbf2, out_ref, prex_ref):
        x2 = seq_ref[...].reshape(T * B, IN)
        prex_ref[...] = lax.dot_general(
            x2, wih0[...], (((1,), (1,)), ((), ())),
            preferred_element_type=jnp.float32).reshape(T, B, 4 * H)
        bias0 = (rbi0[...] + rbh0[...])[None, :]
        bias1 = (rbi1[...] + rbh1[...])[None, :]
        z = jnp.zeros((B, H), jnp.float32)
        h0, c0, h1, c1 = z, z, z, z
        for t in range(T):
            g0 = prex_ref[t] + lax.dot_general(
                h0, whh0[...], (((1,), (1,)), ((), ())),
                preferred_element_type=jnp.float32) + bias0
            i0 = jax.nn.sigmoid(g0[:, :H])
            f0 = jax.nn.sigmoid(g0[:, H:2 * H])
            gg0 = jnp.tanh(g0[:, 2 * H:3 * H])
            o0 = jax.nn.sigmoid(g0[:, 3 * H:])
            c0 = f0 * c0 + i0 * gg0
            h0 = o0 * jnp.tanh(c0)
            g1 = (lax.dot_general(h0, wih1[...], (((1,), (1,)), ((), ())),
                                  preferred_element_type=jnp.float32)
                  + lax.dot_general(h1, whh1[...], (((1,), (1,)), ((), ())),
                                    preferred_element_type=jnp.float32) + bias1)
            i1 = jax.nn.sigmoid(g1[:, :H])
            f1 = jax.nn.sigmoid(g1[:, H:2 * H])
            gg1 = jnp.tanh(g1[:, 2 * H:3 * H])
            o1 = jax.nn.sigmoid(g1[:, 3 * H:])
            c1 = f1 * c1 + i1 * gg1
            h1 = o1 * jnp.tanh(c1)

        aggT = jnp.sum(part_ref[...], axis=0)
        spatial = jax.nn.relu(
            lax.dot_general(aggT, wrel[...], (((0,), (0,)), ((), ())),
                            preferred_element_type=jnp.float32)
            + jnp.dot(bemb_ref[...], wroot[...],
                      preferred_element_type=jnp.float32)
            + rbg[...][None, :])
        W1 = wf1[...]
        hfc = jax.nn.relu(
            jnp.dot(h1, W1[:H], preferred_element_type=jnp.float32)
            + jnp.dot(spatial, W1[H:], preferred_element_type=jnp.float32)
            + rbf1[...][None, :])
        pred = lax.dot_general(wf2[...], hfc, (((0,), (1,)), ((), ())),
                               preferred_element_type=jnp.float32)
        out_ref[...] = pred + rbf2[0]

    out = pl.pallas_call(
        body,
        out_shape=jax.ShapeDtypeStruct((1, B), jnp.float32),
        scratch_shapes=[pltpu.VMEM((T, B, 4 * H), jnp.float32)],
    )(seq_t, partials, bemb, Wih0, Whh0, bi0, bh0, Wih1, Whh1, bi1, bh1,
      Wrel, Wroot, bgn, Wf1, bf1, Wf2, bf2)
    return out.reshape(B)


def kernel(seq_batch, node_idx_batch, edge_index, emb_table,
           W_ih_l0, W_hh_l0, b_ih_l0, b_hh_l0,
           W_ih_l1, W_hh_l1, b_ih_l1, b_hh_l1,
           W_rel, W_root, b_gnn, W_fc1, b_fc1, W_fc2, b_fc2):
    partials, bemb = _sc_edge_filter(node_idx_batch, edge_index[0],
                                     edge_index[1], emb_table)
    seq_t = jnp.swapaxes(seq_batch, 0, 1)
    return _tc_fused(seq_t, partials, bemb,
                     W_ih_l0, W_hh_l0, b_ih_l0, b_hh_l0,
                     W_ih_l1, W_hh_l1, b_ih_l1, b_hh_l1,
                     W_rel, W_root, b_gnn, W_fc1, b_fc1, W_fc2, b_fc2)


# speedup vs baseline: 6.4544x; 1.0005x over previous
import functools

import jax
import jax.numpy as jnp
from jax import lax
from jax.experimental import pallas as pl
from jax.experimental.pallas import tpu as pltpu
from jax.experimental.pallas import tpu_sc as plsc


def _sc_edge_filter(node_idx, edge_src, edge_dst, emb_table):
    info = plsc.get_sparse_core_info()
    NC, NS = info.num_cores, info.num_subcores
    NW = NC * NS
    E = edge_src.shape[0]
    N, D = emb_table.shape
    B = node_idx.shape[0]
    assert E % NW == 0 and (E // NW) % 8 == 0
    EPW = E // NW
    G = (EPW + 15) // 16

    mesh = plsc.VectorSubcoreMesh(core_axis_name="c", subcore_axis_name="s")

    @functools.partial(
        pl.kernel,
        out_type=(
            jax.ShapeDtypeStruct((NW, D, B), jnp.float32),
            jax.ShapeDtypeStruct((B, D), jnp.float32),
        ),
        mesh=mesh,
        compiler_params=pltpu.CompilerParams(needs_layout_passes=False,
                                             use_tc_tiling_on_sc=False),
        scratch_types=[
            pltpu.VMEM((N,), jnp.int32),
            pltpu.VMEM((G * 16,), jnp.int32),
            pltpu.VMEM((G * 16,), jnp.int32),
            pltpu.VMEM((B * D,), jnp.float32),
            pltpu.VMEM((16, D), jnp.float32),
            pltpu.VMEM((B,), jnp.int32),
            pltpu.VMEM((D, B), jnp.float32),
            pltpu.VMEM((B,), jnp.int32),
            pltpu.VMEM((B, D), jnp.float32),
            pltpu.SemaphoreType.DMA,
        ],
    )
    def k(nidx_hbm, src_hbm, dst_hbm, emb_hbm, part_out, bemb_out,
          invmap, dstb, srcb, accf, rows, nidxv, outp, tmpb, bembv, sem):
        cid = lax.axis_index("c")
        sid = lax.axis_index("s")
        wid = sid * NC + cid
        lane = lax.iota(jnp.int32, 16)

        base_e = wid * EPW
        pltpu.sync_copy(src_hbm.at[pl.ds(base_e, EPW)], srcb.at[pl.ds(0, EPW)])
        pltpu.sync_copy(dst_hbm.at[pl.ds(base_e, EPW)], dstb.at[pl.ds(0, EPW)])
        pltpu.sync_copy(nidx_hbm, nidxv)

        @pl.when(wid == 0)
        def _():
            pltpu.async_copy(emb_hbm.at[nidxv], bembv, sem).wait()
            pltpu.sync_copy(bembv, bemb_out)

        with jax.named_scope("sc_memset"):
            def mset(i, _):
                invmap[pl.ds(i * 16, 16)] = jnp.full((16,), -1, jnp.int32)
                return 0
            lax.fori_loop(0, N // 16, mset, 0)

            def aset(i, _):
                accf[pl.ds(i * 16, 16)] = jnp.zeros((16,), jnp.float32)
                return 0
            lax.fori_loop(0, (B * D) // 16, aset, 0)

            for bg in range(B // 16):
                idxv = nidxv[pl.ds(bg * 16, 16)]
                plsc.store_scatter(invmap, [idxv], bg * 16 + lane)

        def group(g, _):
            base = g * 16
            valid = (base + lane) < EPW
            dstv = jnp.where(valid, dstb[pl.ds(base, 16)], 0)
            slots = plsc.load_gather(invmap, [dstv])
            m = (slots >= 0) & valid
            cnt = jnp.sum(m.astype(jnp.int32))

            @pl.when(cnt > 0)
            def _():
                srcv = jnp.where(m, srcb[pl.ds(base, 16)], 0)
                slots0 = jnp.where(m, slots, 0)
                pltpu.async_copy(emb_hbm.at[srcv], rows, sem).wait()
                sbase = slots0 * D

                def cond(rem):
                    return jnp.sum(rem) > 0

                def body(rem):
                    remb = rem > 0
                    plsc.store_scatter(tmpb, [slots0], lane, mask=remb)
                    rb = plsc.load_gather(tmpb, [slots0])
                    sel = (rb == lane) & remb
                    for c in range(D):
                        colv = plsc.load_gather(
                            rows, [lane, jnp.full((16,), c, jnp.int32)])
                        plsc.addupdate_scatter(accf, [sbase + c], colv, mask=sel)
                    return jnp.where(sel, 0, rem)

                lax.while_loop(cond, body, m.astype(jnp.int32))
            return 0
        with jax.named_scope("sc_scan"):
            lax.fori_loop(0, G, group, 0)

        sc_remap = jax.named_scope("sc_remap")
        sc_remap.__enter__()
        for bg in range(B // 16):
            nv = nidxv[pl.ds(bg * 16, 16)]
            slotv = plsc.load_gather(invmap, [nv])
            sb = slotv * D

            def remap(c, _):
                vals = plsc.load_gather(accf, [sb + c])
                outp[c, pl.ds(bg * 16, 16)] = vals
                return 0
            lax.fori_loop(0, D, remap, 0)
        pltpu.sync_copy(outp, part_out.at[wid])
        sc_remap.__exit__(None, None, None)

    return k(node_idx, edge_src, edge_dst, emb_table)


def _tc_fused(seq_t, partials, bemb,
              Wih0, Whh0, bi0, bh0, Wih1, Whh1, bi1, bh1,
              Wrel, Wroot, bgn, Wf1, bf1, Wf2, bf2):
    T, B, IN = seq_t.shape
    H = Whh0.shape[1]

    def body(seq_ref, part_ref, bemb_ref,
             wih0, whh0, rbi0, rbh0, wih1, whh1, rbi1, rbh1,
             wrel, wroot, rbg, wf1, rbf1, wf2, rbf2, out_ref, prex_ref):
        x2 = seq_ref[...].reshape(T * B, IN)
        prex_ref[...] = lax.dot_general(
            x2, wih0[...], (((1,), (1,)), ((), ())),
            preferred_element_type=jnp.float32).reshape(T, B, 4 * H)
        bias0 = (rbi0[...] + rbh0[...])[None, :]
        bias1 = (rbi1[...] + rbh1[...])[None, :]
        z = jnp.zeros((B, H), jnp.float32)
        h0, c0, h1, c1 = z, z, z, z
        for t in range(T):
            g0 = prex_ref[t] + lax.dot_general(
                h0, whh0[...], (((1,), (1,)), ((), ())),
                preferred_element_type=jnp.float32) + bias0
            i0 = jax.nn.sigmoid(g0[:, :H])
            f0 = jax.nn.sigmoid(g0[:, H:2 * H])
            gg0 = jnp.tanh(g0[:, 2 * H:3 * H])
            o0 = jax.nn.sigmoid(g0[:, 3 * H:])
            c0 = f0 * c0 + i0 * gg0
            h0 = o0 * jnp.tanh(c0)
            g1 = (lax.dot_general(h0, wih1[...], (((1,), (1,)), ((), ())),
                                  preferred_element_type=jnp.float32)
                  + lax.dot_general(h1, whh1[...], (((1,), (1,)), ((), ())),
                                    preferred_element_type=jnp.float32) + bias1)
            i1 = jax.nn.sigmoid(g1[:, :H])
            f1 = jax.nn.sigmoid(g1[:, H:2 * H])
            gg1 = jnp.tanh(g1[:, 2 * H:3 * H])
            o1 = jax.nn.sigmoid(g1[:, 3 * H:])
            c1 = f1 * c1 + i1 * gg1
            h1 = o1 * jnp.tanh(c1)

        aggT = jnp.sum(part_ref[...], axis=0)
        spatial = jax.nn.relu(
            lax.dot_general(aggT, wrel[...], (((0,), (0,)), ((), ())),
                            preferred_element_type=jnp.float32)
            + jnp.dot(bemb_ref[...], wroot[...],
                      preferred_element_type=jnp.float32)
            + rbg[...][None, :])
        W1 = wf1[...]
        hfc = jax.nn.relu(
            jnp.dot(h1, W1[:H], preferred_element_type=jnp.float32)
            + jnp.dot(spatial, W1[H:], preferred_element_type=jnp.float32)
            + rbf1[...][None, :])
        pred = lax.dot_general(wf2[...], hfc, (((0,), (1,)), ((), ())),
                               preferred_element_type=jnp.float32)
        out_ref[...] = pred + rbf2[0]

    out = pl.pallas_call(
        body,
        out_shape=jax.ShapeDtypeStruct((1, B), jnp.float32),
        scratch_shapes=[pltpu.VMEM((T, B, 4 * H), jnp.float32)],
    )(seq_t, partials, bemb, Wih0, Whh0, bi0, bh0, Wih1, Whh1, bi1, bh1,
      Wrel, Wroot, bgn, Wf1, bf1, Wf2, bf2)
    return out.reshape(B)


def kernel(seq_batch, node_idx_batch, edge_index, emb_table,
           W_ih_l0, W_hh_l0, b_ih_l0, b_hh_l0,
           W_ih_l1, W_hh_l1, b_ih_l1, b_hh_l1,
           W_rel, W_root, b_gnn, W_fc1, b_fc1, W_fc2, b_fc2):
    partials, bemb = _sc_edge_filter(node_idx_batch, edge_index[0],
                                     edge_index[1], emb_table)
    seq_t = jnp.swapaxes(seq_batch, 0, 1)
    return _tc_fused(seq_t, partials, bemb,
                     W_ih_l0, W_hh_l0, b_ih_l0, b_hh_l0,
                     W_ih_l1, W_hh_l1, b_ih_l1, b_hh_l1,
                     W_rel, W_root, b_gnn, W_fc1, b_fc1, W_fc2, b_fc2)


# speedup vs baseline: 25.9250x; 4.0166x over previous
import functools

import jax
import jax.numpy as jnp
from jax import lax
from jax.experimental import pallas as pl
from jax.experimental.pallas import tpu as pltpu
from jax.experimental.pallas import tpu_sc as plsc


def _sc_edge_filter(node_idx, edge_src, edge_dst, emb_table):
    info = plsc.get_sparse_core_info()
    NC, NS = info.num_cores, info.num_subcores
    NW = NC * NS
    E = edge_src.shape[0]
    N, D = emb_table.shape
    B = node_idx.shape[0]
    assert E % NW == 0 and (E // NW) % 8 == 0
    EPW = E // NW
    G = (EPW + 15) // 16

    mesh = plsc.VectorSubcoreMesh(core_axis_name="c", subcore_axis_name="s")

    SG = 32
    NSG = (G + SG - 1) // SG
    GP = NSG * SG
    CAP = 2048
    DRAIN = CAP - SG * 16
    RB = 64

    @functools.partial(
        pl.kernel,
        out_type=(
            jax.ShapeDtypeStruct((NW, D, B), jnp.float32),
            jax.ShapeDtypeStruct((B, D), jnp.float32),
        ),
        mesh=mesh,
        compiler_params=pltpu.CompilerParams(needs_layout_passes=False,
                                             use_tc_tiling_on_sc=False),
        scratch_types=[
            pltpu.VMEM((N,), jnp.int32),
            pltpu.VMEM((GP * 16,), jnp.int32),
            pltpu.VMEM((GP * 16,), jnp.int32),
            pltpu.VMEM((CAP,), jnp.int32),
            pltpu.VMEM((RB,), jnp.int32),
            pltpu.VMEM((RB, D), jnp.float32),
            pltpu.VMEM((B * D,), jnp.float32),
            pltpu.VMEM((B,), jnp.int32),
            pltpu.VMEM((D, B), jnp.float32),
            pltpu.VMEM((B,), jnp.int32),
            pltpu.SemaphoreType.DMA,
            pltpu.SemaphoreType.DMA,
            pltpu.SemaphoreType.DMA,
        ],
    )
    def k(nidx_hbm, src_hbm, dst_hbm, emb_hbm, part_out, bemb_out,
          invmap, dstb, srcb, comp, idxb, rows, accf, nidxv, outp, tmpb,
          sem, sem2, sem3):
        cid = lax.axis_index("c")
        sid = lax.axis_index("s")
        wid = sid * NC + cid
        lane = lax.iota(jnp.int32, 16)

        base_e = wid * EPW
        cp_src = pltpu.async_copy(src_hbm.at[pl.ds(base_e, EPW)],
                                  srcb.at[pl.ds(0, EPW)], sem)
        cp_dst = pltpu.async_copy(dst_hbm.at[pl.ds(base_e, EPW)],
                                  dstb.at[pl.ds(0, EPW)], sem2)
        pltpu.sync_copy(nidx_hbm, nidxv)

        @pl.when(wid == 0)
        def _():
            for h in range(B // RB):
                pltpu.async_copy(emb_hbm.at[nidxv.at[pl.ds(h * RB, RB)]],
                                 rows, sem3).wait()
                pltpu.sync_copy(rows, bemb_out.at[pl.ds(h * RB, RB)])

        def aset(i, _):
            accf[pl.ds(i * 16, 16)] = jnp.zeros((16,), jnp.float32)
            return 0
        lax.fori_loop(0, (B * D) // 16, aset, 0)

        for bg in range(B // 16):
            idxv = nidxv[pl.ds(bg * 16, 16)]
            plsc.store_scatter(invmap, [idxv], bg * 16 + lane)

        def drain(n):
            nsb = (n + (RB - 1)) // RB

            def sub(sb, _):
                sbase = sb * RB
                for j in range(RB // 16):
                    pos = sbase + j * 16 + lane
                    packv = comp[pl.ds(sbase + j * 16, 16)]
                    srcv = jnp.where(pos < n, packv >> 7, 0)
                    idxb[pl.ds(j * 16, 16)] = srcv
                pltpu.async_copy(emb_hbm.at[idxb], rows, sem).wait()
                for j in range(RB // 16):
                    pos = sbase + j * 16 + lane
                    packv = comp[pl.ds(sbase + j * 16, 16)]
                    slotj = packv & 127
                    mj = pos < n
                    sbj = slotj * D

                    def cond(rem):
                        return jnp.sum(rem) > 0

                    def body(rem):
                        remb = rem > 0
                        plsc.store_scatter(tmpb, [slotj], lane, mask=remb)
                        rb_ = plsc.load_gather(tmpb, [slotj])
                        sel = (rb_ == lane) & remb
                        for c in range(D):
                            colv = plsc.load_gather(
                                rows, [j * 16 + lane,
                                       jnp.full((16,), c, jnp.int32)])
                            plsc.addupdate_scatter(accf, [sbj + c], colv,
                                                   mask=sel)
                        return jnp.where(sel, 0, rem)

                    lax.while_loop(cond, body, mj.astype(jnp.int32))
                return 0
            lax.fori_loop(0, nsb, sub, 0)

        cp_src.wait()
        cp_dst.wait()

        def supergroup(s, offv):
            def group(i, offv):
                base = (s * SG + i) * 16
                valid = (base + lane) < EPW
                dstv = jnp.where(valid, dstb[pl.ds(base, 16)], 0)
                slots = plsc.load_gather(invmap, [dstv]) & 127
                chk = plsc.load_gather(nidxv, [slots])
                m = (chk == dstv) & valid
                srcv = srcb[pl.ds(base, 16)]
                packv = (srcv << 7) | slots
                csum = plsc.cumsum(m.astype(jnp.int32))
                plsc.store_scatter(comp, [offv + csum - 1], packv, mask=m)
                return offv + plsc.all_reduce_population_count(m)
            offv = lax.fori_loop(0, SG, group, offv)
            off = jnp.max(offv)

            @pl.when(off >= DRAIN)
            def _():
                drain(off)
            return jnp.where(offv >= DRAIN, 0, offv)

        with jax.named_scope("sc_scan"):
            offv = lax.fori_loop(0, NSG, supergroup, jnp.zeros((16,), jnp.int32))
        with jax.named_scope("sc_drain"):
            drain(jnp.max(offv))

        sc_remap = jax.named_scope("sc_remap")
        sc_remap.__enter__()
        for bg in range(B // 16):
            nv = nidxv[pl.ds(bg * 16, 16)]
            slotv = plsc.load_gather(invmap, [nv]) & 127
            sb = slotv * D

            def remap(c, _):
                vals = plsc.load_gather(accf, [sb + c])
                outp[c, pl.ds(bg * 16, 16)] = vals
                return 0
            lax.fori_loop(0, D, remap, 0)
        pltpu.sync_copy(outp, part_out.at[wid])
        sc_remap.__exit__(None, None, None)

    return k(node_idx, edge_src, edge_dst, emb_table)


def _tc_fused(seq_t, partials, bemb,
              Wih0, Whh0, bi0, bh0, Wih1, Whh1, bi1, bh1,
              Wrel, Wroot, bgn, Wf1, bf1, Wf2, bf2):
    T, B, IN = seq_t.shape
    H = Whh0.shape[1]

    def body(seq_ref, part_ref, bemb_ref,
             wih0, whh0, rbi0, rbh0, wih1, whh1, rbi1, rbh1,
             wrel, wroot, rbg, wf1, rbf1, wf2, rbf2, out_ref, prex_ref):
        x2 = seq_ref[...].reshape(T * B, IN)
        prex_ref[...] = lax.dot_general(
            x2, wih0[...], (((1,), (1,)), ((), ())),
            preferred_element_type=jnp.float32).reshape(T, B, 4 * H)
        bias0 = (rbi0[...] + rbh0[...])[None, :]
        bias1 = (rbi1[...] + rbh1[...])[None, :]
        z = jnp.zeros((B, H), jnp.float32)
        h0, c0, h1, c1 = z, z, z, z
        for t in range(T):
            g0 = prex_ref[t] + lax.dot_general(
                h0, whh0[...], (((1,), (1,)), ((), ())),
                preferred_element_type=jnp.float32) + bias0
            i0 = jax.nn.sigmoid(g0[:, :H])
            f0 = jax.nn.sigmoid(g0[:, H:2 * H])
            gg0 = jnp.tanh(g0[:, 2 * H:3 * H])
            o0 = jax.nn.sigmoid(g0[:, 3 * H:])
            c0 = f0 * c0 + i0 * gg0
            h0 = o0 * jnp.tanh(c0)
            g1 = (lax.dot_general(h0, wih1[...], (((1,), (1,)), ((), ())),
                                  preferred_element_type=jnp.float32)
                  + lax.dot_general(h1, whh1[...], (((1,), (1,)), ((), ())),
                                    preferred_element_type=jnp.float32) + bias1)
            i1 = jax.nn.sigmoid(g1[:, :H])
            f1 = jax.nn.sigmoid(g1[:, H:2 * H])
            gg1 = jnp.tanh(g1[:, 2 * H:3 * H])
            o1 = jax.nn.sigmoid(g1[:, 3 * H:])
            c1 = f1 * c1 + i1 * gg1
            h1 = o1 * jnp.tanh(c1)

        aggT = jnp.sum(part_ref[...], axis=0)
        spatial = jax.nn.relu(
            lax.dot_general(aggT, wrel[...], (((0,), (0,)), ((), ())),
                            preferred_element_type=jnp.float32)
            + jnp.dot(bemb_ref[...], wroot[...],
                      preferred_element_type=jnp.float32)
            + rbg[...][None, :])
        W1 = wf1[...]
        hfc = jax.nn.relu(
            jnp.dot(h1, W1[:H], preferred_element_type=jnp.float32)
            + jnp.dot(spatial, W1[H:], preferred_element_type=jnp.float32)
            + rbf1[...][None, :])
        pred = lax.dot_general(wf2[...], hfc, (((0,), (1,)), ((), ())),
                               preferred_element_type=jnp.float32)
        out_ref[...] = pred + rbf2[0]

    out = pl.pallas_call(
        body,
        out_shape=jax.ShapeDtypeStruct((1, B), jnp.float32),
        scratch_shapes=[pltpu.VMEM((T, B, 4 * H), jnp.float32)],
    )(seq_t, partials, bemb, Wih0, Whh0, bi0, bh0, Wih1, Whh1, bi1, bh1,
      Wrel, Wroot, bgn, Wf1, bf1, Wf2, bf2)
    return out.reshape(B)


def kernel(seq_batch, node_idx_batch, edge_index, emb_table,
           W_ih_l0, W_hh_l0, b_ih_l0, b_hh_l0,
           W_ih_l1, W_hh_l1, b_ih_l1, b_hh_l1,
           W_rel, W_root, b_gnn, W_fc1, b_fc1, W_fc2, b_fc2):
    partials, bemb = _sc_edge_filter(node_idx_batch, edge_index[0],
                                     edge_index[1], emb_table)
    seq_t = jnp.swapaxes(seq_batch, 0, 1)
    return _tc_fused(seq_t, partials, bemb,
                     W_ih_l0, W_hh_l0, b_ih_l0, b_hh_l0,
                     W_ih_l1, W_hh_l1, b_ih_l1, b_hh_l1,
                     W_rel, W_root, b_gnn, W_fc1, b_fc1, W_fc2, b_fc2)


# speedup vs baseline: 28.7434x; 1.1087x over previous
import functools

import jax
import jax.numpy as jnp
from jax import lax
from jax.experimental import pallas as pl
from jax.experimental.pallas import tpu as pltpu
from jax.experimental.pallas import tpu_sc as plsc


def _sc_edge_filter(node_idx, edge_index, emb_table):
    info = plsc.get_sparse_core_info()
    NC, NS = info.num_cores, info.num_subcores
    NW = NC * NS
    E = edge_index.shape[1]
    N, D = emb_table.shape
    B = node_idx.shape[0]
    assert E % NW == 0 and (E // NW) % 8 == 0
    EPW = E // NW
    G = (EPW + 15) // 16

    mesh = plsc.VectorSubcoreMesh(core_axis_name="c", subcore_axis_name="s")

    SG = 32
    NSG = (G + SG - 1) // SG
    GP = NSG * SG
    CAP = 2048
    DRAIN = CAP - SG * 16
    RB = 64

    @functools.partial(
        pl.kernel,
        out_type=(
            jax.ShapeDtypeStruct((NW, D, B), jnp.float32),
            jax.ShapeDtypeStruct((B, D), jnp.float32),
        ),
        mesh=mesh,
        compiler_params=pltpu.CompilerParams(needs_layout_passes=False,
                                             use_tc_tiling_on_sc=False),
        scratch_types=[
            pltpu.VMEM((N,), jnp.int32),
            pltpu.VMEM((GP * 16,), jnp.int32),
            pltpu.VMEM((GP * 16,), jnp.int32),
            pltpu.VMEM((CAP,), jnp.int32),
            pltpu.VMEM((RB,), jnp.int32),
            pltpu.VMEM((RB, D), jnp.float32),
            pltpu.VMEM((B * D,), jnp.float32),
            pltpu.VMEM((B,), jnp.int32),
            pltpu.VMEM((D, B), jnp.float32),
            pltpu.VMEM((B,), jnp.int32),
            pltpu.SemaphoreType.DMA,
            pltpu.SemaphoreType.DMA,
            pltpu.SemaphoreType.DMA,
        ],
    )
    def k(nidx_hbm, edge_hbm, emb_hbm, part_out, bemb_out,
          invmap, dstb, srcb, comp, idxb, rows, accf, nidxv, outp, tmpb,
          sem, sem2, sem3):
        cid = lax.axis_index("c")
        sid = lax.axis_index("s")
        wid = sid * NC + cid
        lane = lax.iota(jnp.int32, 16)

        base_e = wid * EPW
        cp_src = pltpu.async_copy(edge_hbm.at[0, pl.ds(base_e, EPW)],
                                  srcb.at[pl.ds(0, EPW)], sem)
        cp_dst = pltpu.async_copy(edge_hbm.at[1, pl.ds(base_e, EPW)],
                                  dstb.at[pl.ds(0, EPW)], sem2)
        pltpu.sync_copy(nidx_hbm, nidxv)

        @pl.when(wid < B // RB)
        def _():
            pltpu.async_copy(emb_hbm.at[nidxv.at[pl.ds(wid * RB, RB)]],
                             rows, sem3).wait()
            pltpu.sync_copy(rows, bemb_out.at[pl.ds(wid * RB, RB)])

        def aset(i, _):
            accf[pl.ds(i * 16, 16)] = jnp.zeros((16,), jnp.float32)
            return 0
        lax.fori_loop(0, (B * D) // 16, aset, 0)

        for bg in range(B // 16):
            idxv = nidxv[pl.ds(bg * 16, 16)]
            plsc.store_scatter(invmap, [idxv], bg * 16 + lane)

        def drain(n):
            nsb = (n + (RB - 1)) // RB

            def sub(sb, _):
                sbase = sb * RB
                for j in range(RB // 16):
                    pos = sbase + j * 16 + lane
                    packv = comp[pl.ds(sbase + j * 16, 16)]
                    srcv = jnp.where(pos < n, packv >> 7, 0)
                    idxb[pl.ds(j * 16, 16)] = srcv
                pltpu.async_copy(emb_hbm.at[idxb], rows, sem).wait()
                for j in range(RB // 16):
                    pos = sbase + j * 16 + lane
                    packv = comp[pl.ds(sbase + j * 16, 16)]
                    slotj = packv & 127
                    mj = pos < n
                    sbj = slotj * D

                    def cond(rem):
                        return jnp.sum(rem) > 0

                    def body(rem):
                        remb = rem > 0
                        plsc.store_scatter(tmpb, [slotj], lane, mask=remb)
                        rb_ = plsc.load_gather(tmpb, [slotj])
                        sel = (rb_ == lane) & remb
                        for c in range(D):
                            colv = plsc.load_gather(
                                rows, [j * 16 + lane,
                                       jnp.full((16,), c, jnp.int32)])
                            plsc.addupdate_scatter(accf, [sbj + c], colv,
                                                   mask=sel)
                        return jnp.where(sel, 0, rem)

                    lax.while_loop(cond, body, mj.astype(jnp.int32))
                return 0
            lax.fori_loop(0, nsb, sub, 0)

        cp_src.wait()
        cp_dst.wait()

        def supergroup(s, offv):
            def group(i, offv):
                base = (s * SG + i) * 16
                valid = (base + lane) < EPW
                dstv = jnp.where(valid, dstb[pl.ds(base, 16)], 0)
                slots = plsc.load_gather(invmap, [dstv]) & 127
                chk = plsc.load_gather(nidxv, [slots])
                m = (chk == dstv) & valid
                srcv = srcb[pl.ds(base, 16)]
                packv = (srcv << 7) | slots
                csum = plsc.cumsum(m.astype(jnp.int32))
                plsc.store_scatter(comp, [offv + csum - 1], packv, mask=m)
                return offv + plsc.all_reduce_population_count(m)
            offv = lax.fori_loop(0, SG, group, offv)
            off = jnp.max(offv)

            @pl.when(off >= DRAIN)
            def _():
                drain(off)
            return jnp.where(offv >= DRAIN, 0, offv)

        with jax.named_scope("sc_scan"):
            offv = lax.fori_loop(0, NSG, supergroup, jnp.zeros((16,), jnp.int32))
        with jax.named_scope("sc_drain"):
            drain(jnp.max(offv))

        sc_remap = jax.named_scope("sc_remap")
        sc_remap.__enter__()
        for bg in range(B // 16):
            nv = nidxv[pl.ds(bg * 16, 16)]
            slotv = plsc.load_gather(invmap, [nv]) & 127
            sb = slotv * D

            def remap(c, _):
                vals = plsc.load_gather(accf, [sb + c])
                outp[c, pl.ds(bg * 16, 16)] = vals
                return 0
            lax.fori_loop(0, D, remap, 0)
        pltpu.sync_copy(outp, part_out.at[wid])
        sc_remap.__exit__(None, None, None)

    return k(node_idx, edge_index, emb_table)


def _tc_fused(seq_t, partials, bemb,
              Wih0, Whh0, bi0, bh0, Wih1, Whh1, bi1, bh1,
              Wrel, Wroot, bgn, Wf1, bf1, Wf2, bf2):
    T, B, IN = seq_t.shape
    H = Whh0.shape[1]

    def body(seq_ref, part_ref, bemb_ref,
             wih0, whh0, rbi0, rbh0, wih1, whh1, rbi1, rbh1,
             wrel, wroot, rbg, wf1, rbf1, wf2, rbf2, out_ref, prex_ref):
        x2 = seq_ref[...].reshape(T * B, IN)
        prex_ref[...] = lax.dot_general(
            x2, wih0[...], (((1,), (1,)), ((), ())),
            preferred_element_type=jnp.float32).reshape(T, B, 4 * H)
        bias0 = (rbi0[...] + rbh0[...])[None, :]
        bias1 = (rbi1[...] + rbh1[...])[None, :]
        z = jnp.zeros((B, H), jnp.float32)
        h0, c0, h1, c1 = z, z, z, z
        for t in range(T):
            g0 = prex_ref[t] + lax.dot_general(
                h0, whh0[...], (((1,), (1,)), ((), ())),
                preferred_element_type=jnp.float32) + bias0
            i0 = jax.nn.sigmoid(g0[:, :H])
            f0 = jax.nn.sigmoid(g0[:, H:2 * H])
            gg0 = jnp.tanh(g0[:, 2 * H:3 * H])
            o0 = jax.nn.sigmoid(g0[:, 3 * H:])
            c0 = f0 * c0 + i0 * gg0
            h0 = o0 * jnp.tanh(c0)
            g1 = (lax.dot_general(h0, wih1[...], (((1,), (1,)), ((), ())),
                                  preferred_element_type=jnp.float32)
                  + lax.dot_general(h1, whh1[...], (((1,), (1,)), ((), ())),
                                    preferred_element_type=jnp.float32) + bias1)
            i1 = jax.nn.sigmoid(g1[:, :H])
            f1 = jax.nn.sigmoid(g1[:, H:2 * H])
            gg1 = jnp.tanh(g1[:, 2 * H:3 * H])
            o1 = jax.nn.sigmoid(g1[:, 3 * H:])
            c1 = f1 * c1 + i1 * gg1
            h1 = o1 * jnp.tanh(c1)

        aggT = jnp.sum(part_ref[...], axis=0)
        spatial = jax.nn.relu(
            lax.dot_general(aggT, wrel[...], (((0,), (0,)), ((), ())),
                            preferred_element_type=jnp.float32)
            + jnp.dot(bemb_ref[...], wroot[...],
                      preferred_element_type=jnp.float32)
            + rbg[...][None, :])
        W1 = wf1[...]
        hfc = jax.nn.relu(
            jnp.dot(h1, W1[:H], preferred_element_type=jnp.float32)
            + jnp.dot(spatial, W1[H:], preferred_element_type=jnp.float32)
            + rbf1[...][None, :])
        pred = lax.dot_general(wf2[...], hfc, (((0,), (1,)), ((), ())),
                               preferred_element_type=jnp.float32)
        out_ref[...] = pred + rbf2[0]

    out = pl.pallas_call(
        body,
        out_shape=jax.ShapeDtypeStruct((1, B), jnp.float32),
        scratch_shapes=[pltpu.VMEM((T, B, 4 * H), jnp.float32)],
    )(seq_t, partials, bemb, Wih0, Whh0, bi0, bh0, Wih1, Whh1, bi1, bh1,
      Wrel, Wroot, bgn, Wf1, bf1, Wf2, bf2)
    return out.reshape(B)


def kernel(seq_batch, node_idx_batch, edge_index, emb_table,
           W_ih_l0, W_hh_l0, b_ih_l0, b_hh_l0,
           W_ih_l1, W_hh_l1, b_ih_l1, b_hh_l1,
           W_rel, W_root, b_gnn, W_fc1, b_fc1, W_fc2, b_fc2):
    partials, bemb = _sc_edge_filter(node_idx_batch, edge_index, emb_table)
    seq_t = jnp.swapaxes(seq_batch, 0, 1)
    return _tc_fused(seq_t, partials, bemb,
                     W_ih_l0, W_hh_l0, b_ih_l0, b_hh_l0,
                     W_ih_l1, W_hh_l1, b_ih_l1, b_hh_l1,
                     W_rel, W_root, b_gnn, W_fc1, b_fc1, W_fc2, b_fc2)


# speedup vs baseline: 44.1153x; 1.5348x over previous
import functools

import jax
import jax.numpy as jnp
from jax import lax
from jax.experimental import pallas as pl
from jax.experimental.pallas import tpu as pltpu
from jax.experimental.pallas import tpu_sc as plsc

_SC_PARAMS = pltpu.CompilerParams(needs_layout_passes=False,
                                  use_tc_tiling_on_sc=False)


def _sc_info():
    info = plsc.get_sparse_core_info()
    return info.num_cores, info.num_subcores


def _sc_scan(node_idx, edge_index):
    NC, NS = _sc_info()
    NW = NC * NS
    E = edge_index.shape[1]
    B = node_idx.shape[0]
    N_CAP = 128
    assert B == N_CAP and E % NW == 0 and (E // NW) % 8 == 0
    EPW = E // NW
    G = (EPW + 15) // 16
    CAP = G * 16 + 16

    mesh = plsc.VectorSubcoreMesh(core_axis_name="c", subcore_axis_name="s")

    @functools.partial(
        pl.kernel,
        out_type=(
            jax.ShapeDtypeStruct((NW, CAP), jnp.int32),
            jax.ShapeDtypeStruct((NW, 16), jnp.int32),
        ),
        mesh=mesh,
        compiler_params=_SC_PARAMS,
        scratch_types=[
            pltpu.VMEM((N_NODES_,), jnp.int32),
            pltpu.VMEM((G * 16,), jnp.int32),
            pltpu.VMEM((G * 16,), jnp.int32),
            pltpu.VMEM((CAP,), jnp.int32),
            pltpu.VMEM((B,), jnp.int32),
            pltpu.VMEM((16,), jnp.int32),
            pltpu.SemaphoreType.DMA,
            pltpu.SemaphoreType.DMA,
        ],
    )
    def k(nidx_hbm, edge_hbm, comp_out, cnt_out,
          invmap, dstb, srcb, comp, nidxv, cntb, sem, sem2):
        cid = lax.axis_index("c")
        sid = lax.axis_index("s")
        wid = sid * NC + cid
        lane = lax.iota(jnp.int32, 16)

        base_e = wid * EPW
        cp_src = pltpu.async_copy(edge_hbm.at[0, pl.ds(base_e, EPW)],
                                  srcb.at[pl.ds(0, EPW)], sem)
        cp_dst = pltpu.async_copy(edge_hbm.at[1, pl.ds(base_e, EPW)],
                                  dstb.at[pl.ds(0, EPW)], sem2)
        pltpu.sync_copy(nidx_hbm, nidxv)
        for bg in range(B // 16):
            idxv = nidxv[pl.ds(bg * 16, 16)]
            plsc.store_scatter(invmap, [idxv], bg * 16 + lane)
        cp_src.wait()
        cp_dst.wait()

        def group(g, offv):
            base = g * 16
            valid = (base + lane) < EPW
            dstv = jnp.where(valid, dstb[pl.ds(base, 16)], 0)
            slots = plsc.load_gather(invmap, [dstv]) & 127
            chk = plsc.load_gather(nidxv, [slots])
            m = (chk == dstv) & valid
            srcv = srcb[pl.ds(base, 16)]
            packv = (srcv << 7) | slots
            csum = plsc.cumsum(m.astype(jnp.int32))
            plsc.store_scatter(comp, [offv + csum - 1], packv, mask=m)
            return offv + plsc.all_reduce_population_count(m)

        with jax.named_scope("sc_scan"):
            offv = plsc.parallel_loop(0, G, carry=jnp.zeros((16,), jnp.int32))(group)

        cntb[...] = offv
        pltpu.sync_copy(comp, comp_out.at[wid])
        pltpu.sync_copy(cntb, cnt_out.at[wid])

    return k(node_idx, edge_index)


def _sc_drain(node_idx, emb_table, comp_hbm, cnt_hbm):
    NC, NS = _sc_info()
    NW = NC * NS
    N, D = emb_table.shape
    B = node_idx.shape[0]
    CAP = comp_hbm.shape[1]
    CH = 2048
    RB = 64

    mesh = plsc.VectorSubcoreMesh(core_axis_name="c", subcore_axis_name="s")

    @functools.partial(
        pl.kernel,
        out_type=(
            jax.ShapeDtypeStruct((NW, D, B), jnp.float32),
            jax.ShapeDtypeStruct((B, D), jnp.float32),
        ),
        mesh=mesh,
        compiler_params=_SC_PARAMS,
        scratch_types=[
            pltpu.VMEM((N,), jnp.int32),
            pltpu.VMEM((CH,), jnp.int32),
            pltpu.VMEM((RB,), jnp.int32),
            pltpu.VMEM((RB, D), jnp.float32),
            pltpu.VMEM((B * D,), jnp.float32),
            pltpu.VMEM((B,), jnp.int32),
            pltpu.VMEM((D, B), jnp.float32),
            pltpu.VMEM((B,), jnp.int32),
            pltpu.VMEM((16,), jnp.int32),
            pltpu.SemaphoreType.DMA,
            pltpu.SemaphoreType.DMA,
        ],
    )
    def k(nidx_hbm, emb_hbm, comp_in, cnt_in, part_out, bemb_out,
          invmap, comps, idxb, rows, accf, nidxv, outp, tmpb, cntb,
          sem, sem2):
        cid = lax.axis_index("c")
        sid = lax.axis_index("s")
        wid = sid * NC + cid
        lane = lax.iota(jnp.int32, 16)

        pltpu.sync_copy(nidx_hbm, nidxv)
        pltpu.sync_copy(cnt_in.at[wid], cntb)
        for bg in range(B // 16):
            idxv = nidxv[pl.ds(bg * 16, 16)]
            plsc.store_scatter(invmap, [idxv], bg * 16 + lane)

        @pl.when(wid < B // RB)
        def _():
            pltpu.async_copy(emb_hbm.at[nidxv.at[pl.ds(wid * RB, RB)]],
                             rows, sem2).wait()
            pltpu.sync_copy(rows, bemb_out.at[pl.ds(wid * RB, RB)])

        @plsc.parallel_loop(0, (B * D) // 16)
        def _(i):
            accf[pl.ds(i * 16, 16)] = jnp.zeros((16,), jnp.float32)

        cnt = jnp.max(cntb[...])

        with jax.named_scope("sc_drain"):
            def chunk(ch, _):
                pltpu.sync_copy(comp_in.at[wid, pl.ds(ch * CH, CH)], comps)
                n = jnp.minimum(cnt - ch * CH, CH)

                def sub(sb, _):
                    sbase = sb * RB
                    for j in range(RB // 16):
                        pos = sbase + j * 16 + lane
                        packv = comps[pl.ds(sbase + j * 16, 16)]
                        srcv = jnp.where(pos < n, packv >> 7, 0)
                        idxb[pl.ds(j * 16, 16)] = srcv
                    pltpu.async_copy(emb_hbm.at[idxb], rows, sem).wait()
                    for j in range(RB // 16):
                        pos = sbase + j * 16 + lane
                        packv = comps[pl.ds(sbase + j * 16, 16)]
                        slotj = packv & 127
                        mj = pos < n
                        sbj = slotj * D

                        def cond(rem):
                            return jnp.sum(rem) > 0

                        def body(rem):
                            remb = rem > 0
                            plsc.store_scatter(tmpb, [slotj], lane, mask=remb)
                            rb_ = plsc.load_gather(tmpb, [slotj])
                            sel = (rb_ == lane) & remb
                            for c in range(D):
                                colv = plsc.load_gather(
                                    rows, [j * 16 + lane,
                                           jnp.full((16,), c, jnp.int32)])
                                plsc.addupdate_scatter(accf, [sbj + c], colv,
                                                       mask=sel)
                            return jnp.where(sel, 0, rem)

                        lax.while_loop(cond, body, mj.astype(jnp.int32))
                    return 0
                lax.fori_loop(0, (n + (RB - 1)) // RB, sub, 0)
                return 0
            lax.fori_loop(0, (cnt + (CH - 1)) // CH, chunk, 0)

        sc_remap = jax.named_scope("sc_remap")
        sc_remap.__enter__()
        for bg in range(B // 16):
            nv = nidxv[pl.ds(bg * 16, 16)]
            slotv = plsc.load_gather(invmap, [nv]) & 127
            sb = slotv * D

            @plsc.parallel_loop(0, D)
            def _(c):
                vals = plsc.load_gather(accf, [sb + c])
                outp[c, pl.ds(bg * 16, 16)] = vals
        pltpu.sync_copy(outp, part_out.at[wid])
        sc_remap.__exit__(None, None, None)

    return k(node_idx, emb_table, comp_hbm, cnt_hbm)


N_NODES_ = 50000


def _tc_lstm(seq_t, Wih0, Whh0, bi0, bh0, Wih1, Whh1, bi1, bh1):
    T, B, IN = seq_t.shape
    H = Whh0.shape[1]

    def body(seq_ref, wih0, whh0, rbi0, rbh0, wih1, whh1, rbi1, rbh1,
             out_ref, prex_ref):
        x2 = seq_ref[...].reshape(T * B, IN)
        prex_ref[...] = lax.dot_general(
            x2, wih0[...], (((1,), (1,)), ((), ())),
            preferred_element_type=jnp.float32).reshape(T, B, 4 * H)
        bias0 = (rbi0[...] + rbh0[...])[None, :]
        bias1 = (rbi1[...] + rbh1[...])[None, :]
        z = jnp.zeros((B, H), jnp.float32)
        h0, c0, h1, c1 = z, z, z, z
        for t in range(T):
            g0 = prex_ref[t] + lax.dot_general(
                h0, whh0[...], (((1,), (1,)), ((), ())),
                preferred_element_type=jnp.float32) + bias0
            i0 = jax.nn.sigmoid(g0[:, :H])
            f0 = jax.nn.sigmoid(g0[:, H:2 * H])
            gg0 = jnp.tanh(g0[:, 2 * H:3 * H])
            o0 = jax.nn.sigmoid(g0[:, 3 * H:])
            c0 = f0 * c0 + i0 * gg0
            h0 = o0 * jnp.tanh(c0)
            g1 = (lax.dot_general(h0, wih1[...], (((1,), (1,)), ((), ())),
                                  preferred_element_type=jnp.float32)
                  + lax.dot_general(h1, whh1[...], (((1,), (1,)), ((), ())),
                                    preferred_element_type=jnp.float32) + bias1)
            i1 = jax.nn.sigmoid(g1[:, :H])
            f1 = jax.nn.sigmoid(g1[:, H:2 * H])
            gg1 = jnp.tanh(g1[:, 2 * H:3 * H])
            o1 = jax.nn.sigmoid(g1[:, 3 * H:])
            c1 = f1 * c1 + i1 * gg1
            h1 = o1 * jnp.tanh(c1)
        out_ref[...] = h1

    return pl.pallas_call(
        body,
        out_shape=jax.ShapeDtypeStruct((B, H), jnp.float32),
        scratch_shapes=[pltpu.VMEM((T, B, 4 * H), jnp.float32)],
    )(seq_t, Wih0, Whh0, bi0, bh0, Wih1, Whh1, bi1, bh1)


def _tc_head(h1, partials, bemb, Wrel, Wroot, bgn, Wf1, bf1, Wf2, bf2):
    B, H = h1.shape

    def body(h1_ref, part_ref, bemb_ref, wrel, wroot, rbg, wf1, rbf1,
             wf2, rbf2, out_ref):
        aggT = jnp.sum(part_ref[...], axis=0)
        spatial = jax.nn.relu(
            lax.dot_general(aggT, wrel[...], (((0,), (0,)), ((), ())),
                            preferred_element_type=jnp.float32)
            + jnp.dot(bemb_ref[...], wroot[...],
                      preferred_element_type=jnp.float32)
            + rbg[...][None, :])
        W1 = wf1[...]
        hfc = jax.nn.relu(
            jnp.dot(h1_ref[...], W1[:H], preferred_element_type=jnp.float32)
            + jnp.dot(spatial, W1[H:], preferred_element_type=jnp.float32)
            + rbf1[...][None, :])
        pred = lax.dot_general(wf2[...], hfc, (((0,), (1,)), ((), ())),
                               preferred_element_type=jnp.float32)
        out_ref[...] = pred + rbf2[0]

    return pl.pallas_call(
        body,
        out_shape=jax.ShapeDtypeStruct((1, B), jnp.float32),
    )(h1, partials, bemb, Wrel, Wroot, bgn, Wf1, bf1, Wf2, bf2)


def kernel(seq_batch, node_idx_batch, edge_index, emb_table,
           W_ih_l0, W_hh_l0, b_ih_l0, b_hh_l0,
           W_ih_l1, W_hh_l1, b_ih_l1, b_hh_l1,
           W_rel, W_root, b_gnn, W_fc1, b_fc1, W_fc2, b_fc2):
    comp, cnt = _sc_scan(node_idx_batch, edge_index)
    seq_t = jnp.swapaxes(seq_batch, 0, 1)
    h1 = _tc_lstm(seq_t, W_ih_l0, W_hh_l0, b_ih_l0, b_hh_l0,
                  W_ih_l1, W_hh_l1, b_ih_l1, b_hh_l1)
    partials, bemb = _sc_drain(node_idx_batch, emb_table, comp, cnt)
    pred = _tc_head(h1, partials, bemb, W_rel, W_root, b_gnn,
                    W_fc1, b_fc1, W_fc2, b_fc2)
    return pred.reshape(seq_batch.shape[0])
